# initial kernel scaffold (unmeasured)
import jax
import jax.numpy as jnp
from jax import lax
from jax.experimental import pallas as pl
from jax.experimental.pallas import tpu as pltpu

N_DEV = 32
M = 2048
D = 512
H = 1024
E_LOCAL = 4
CH = M // N_DEV


def kernel(x, router_W, route_idx, expert_W, shared_W):
    def body(x_ref, rw_ref, idx_ref, ew_ref, sw_ref, out_ref,
             partial_ref, stage_ref, recv_ref,
             rs_send, rs_recv, ag_send, ag_recv):
        my = lax.axis_index("i")
        left = (my + N_DEV - 1) % N_DEV
        right = (my + 1) % N_DEV

        barrier = pltpu.get_barrier_semaphore()
        for nbr in (left, right):
            pl.semaphore_signal(
                barrier, inc=1,
                device_id=(nbr,), device_id_type=pl.DeviceIdType.MESH,
            )
        pl.semaphore_wait(barrier, 2)

        xv = x_ref[:, :]
        scores = jnp.dot(xv, rw_ref[:, :], preferred_element_type=jnp.float32)
        s_max = jnp.max(scores, axis=-1, keepdims=True)
        ex = jnp.exp(scores - s_max)
        probs = ex / jnp.sum(ex, axis=-1, keepdims=True)
        route = idx_ref[:, :]

        acc = jnp.zeros((M, H), jnp.float32)
        for e in range(E_LOCAL):
            ge = my * E_LOCAL + e
            g = lax.dynamic_slice_in_dim(probs, ge, 1, axis=1)
            g = jnp.where(route == ge, g, 0.0)
            acc = acc + jnp.dot(xv * g, ew_ref[e],
                                preferred_element_type=jnp.float32)
        partial_ref[:, :] = acc

        for h in range(N_DEV - 1):
            c = (my + N_DEV - h) % N_DEV
            slot = h % 2
            if h == 0:
                stage_ref[slot, :, :] = partial_ref[pl.ds(c * CH, CH), :]
            else:
                stage_ref[slot, :, :] = (
                    partial_ref[pl.ds(c * CH, CH), :] + recv_ref[h - 1]
                )
            rdma = pltpu.make_async_remote_copy(
                src_ref=stage_ref.at[slot],
                dst_ref=recv_ref.at[h],
                send_sem=rs_send.at[h],
                recv_sem=rs_recv.at[h],
                device_id=(right,),
                device_id_type=pl.DeviceIdType.MESH,
            )
            rdma.start()
            rdma.wait()

        o = (my + 1) % N_DEV
        xo = x_ref[pl.ds(o * CH, CH), :]
        owned = (
            partial_ref[pl.ds(o * CH, CH), :]
            + recv_ref[N_DEV - 2]
            + jnp.dot(xo, sw_ref[:, :], preferred_element_type=jnp.float32)
        )
        out_ref[pl.ds(o * CH, CH), :] = owned

        for h in range(N_DEV - 1):
            src_c = (o + N_DEV - h) % N_DEV
            dst_c = (o + N_DEV - h - 1) % N_DEV
            rdma = pltpu.make_async_remote_copy(
                src_ref=out_ref.at[pl.ds(src_c * CH, CH), :],
                dst_ref=out_ref.at[pl.ds(dst_c * CH, CH), :],
                send_sem=ag_send.at[h],
                recv_sem=ag_recv.at[h],
                device_id=(right,),
                device_id_type=pl.DeviceIdType.MESH,
            )
            rdma.start()
            rdma.wait()

    return pl.pallas_call(
        body,
        out_shape=jax.ShapeDtypeStruct((M, H), jnp.float32),
        in_specs=[
            pl.BlockSpec(memory_space=pltpu.VMEM),
            pl.BlockSpec(memory_space=pltpu.VMEM),
            pl.BlockSpec(memory_space=pltpu.VMEM),
            pl.BlockSpec(memory_space=pltpu.VMEM),
            pl.BlockSpec(memory_space=pltpu.VMEM),
        ],
        out_specs=pl.BlockSpec(memory_space=pltpu.VMEM),
        scratch_shapes=[
            pltpu.VMEM((M, H), jnp.float32),
            pltpu.VMEM((2, CH, H), jnp.float32),
            pltpu.VMEM((N_DEV - 1, CH, H), jnp.float32),
            pltpu.SemaphoreType.DMA((N_DEV - 1,)),
            pltpu.SemaphoreType.DMA((N_DEV - 1,)),
            pltpu.SemaphoreType.DMA((N_DEV - 1,)),
            pltpu.SemaphoreType.DMA((N_DEV - 1,)),
        ],
        compiler_params=pltpu.CompilerParams(collective_id=0),
    )(x, router_W, route_idx, expert_W, shared_W)


# baseline (device time: 323480 ns/iter reference)
import jax
import jax.numpy as jnp
from jax import lax
from jax.experimental import pallas as pl
from jax.experimental.pallas import tpu as pltpu

N_DEV = 32
M = 2048
D = 512
H = 1024
E_LOCAL = 4
CH = M // N_DEV


def kernel(x, router_W, route_idx, expert_W, shared_W):
    def body(x_ref, rw_ref, idx_ref, ew_ref, sw_ref, out_ref,
             partial_ref, stage_ref, recv_ref, g_ref,
             rs_send, rs_recv, ag_send, ag_recv):
        my = lax.axis_index("i")
        left = (my + N_DEV - 1) % N_DEV
        right = (my + 1) % N_DEV

        barrier = pltpu.get_barrier_semaphore()
        for nbr in (left, right):
            pl.semaphore_signal(
                barrier, inc=1,
                device_id=(nbr,), device_id_type=pl.DeviceIdType.MESH,
            )
        pl.semaphore_wait(barrier, 2)

        xv = x_ref[:, :]
        scores = jnp.dot(xv, rw_ref[:, :], preferred_element_type=jnp.float32)
        s_max = jnp.max(scores, axis=-1, keepdims=True)
        ex = jnp.exp(scores - s_max)
        probs = ex / jnp.sum(ex, axis=-1, keepdims=True)
        route = idx_ref[:, :]
        col_ids = lax.broadcasted_iota(jnp.int32, scores.shape, 1)

        acc = jnp.zeros((M, H), jnp.float32)
        for e in range(E_LOCAL):
            ge = my * E_LOCAL + e
            g = jnp.sum(jnp.where(col_ids == ge, probs, 0.0),
                        axis=1, keepdims=True)
            g = jnp.where(route == ge, g, 0.0)
            acc = acc + jnp.dot(xv * g, ew_ref[e],
                                preferred_element_type=jnp.float32)
        partial_ref[:, :] = acc

        for h in range(N_DEV - 1):
            c = (my + N_DEV - h) % N_DEV
            slot = h % 2
            if h == 0:
                stage_ref[slot, :, :] = partial_ref[pl.ds(c * CH, CH), :]
            else:
                stage_ref[slot, :, :] = (
                    partial_ref[pl.ds(c * CH, CH), :] + recv_ref[h - 1]
                )
            rdma = pltpu.make_async_remote_copy(
                src_ref=stage_ref.at[slot],
                dst_ref=recv_ref.at[h],
                send_sem=rs_send.at[h],
                recv_sem=rs_recv.at[h],
                device_id=(right,),
                device_id_type=pl.DeviceIdType.MESH,
            )
            rdma.start()
            rdma.wait()

        o = (my + 1) % N_DEV
        xo = x_ref[pl.ds(o * CH, CH), :]
        g_ref[0, :, :] = (
            partial_ref[pl.ds(o * CH, CH), :]
            + recv_ref[N_DEV - 2]
            + jnp.dot(xo, sw_ref[:, :], preferred_element_type=jnp.float32)
        )

        for h in range(N_DEV - 1):
            rdma = pltpu.make_async_remote_copy(
                src_ref=g_ref.at[h],
                dst_ref=g_ref.at[h + 1],
                send_sem=ag_send.at[h],
                recv_sem=ag_recv.at[h],
                device_id=(right,),
                device_id_type=pl.DeviceIdType.MESH,
            )
            rdma.start()
            rdma.wait()

        for s in range(N_DEV):
            c = (my + 1 + N_DEV - s) % N_DEV
            out_ref[pl.ds(c * CH, CH), :] = g_ref[s, :, :]

    return pl.pallas_call(
        body,
        out_shape=jax.ShapeDtypeStruct((M, H), jnp.float32),
        in_specs=[
            pl.BlockSpec(memory_space=pltpu.VMEM),
            pl.BlockSpec(memory_space=pltpu.VMEM),
            pl.BlockSpec(memory_space=pltpu.VMEM),
            pl.BlockSpec(memory_space=pltpu.VMEM),
            pl.BlockSpec(memory_space=pltpu.VMEM),
        ],
        out_specs=pl.BlockSpec(memory_space=pltpu.VMEM),
        scratch_shapes=[
            pltpu.VMEM((M, H), jnp.float32),
            pltpu.VMEM((2, CH, H), jnp.float32),
            pltpu.VMEM((N_DEV - 1, CH, H), jnp.float32),
            pltpu.VMEM((N_DEV, CH, H), jnp.float32),
            pltpu.SemaphoreType.DMA((N_DEV - 1,)),
            pltpu.SemaphoreType.DMA((N_DEV - 1,)),
            pltpu.SemaphoreType.DMA((N_DEV - 1,)),
            pltpu.SemaphoreType.DMA((N_DEV - 1,)),
        ],
        compiler_params=pltpu.CompilerParams(
            collective_id=0, vmem_limit_bytes=100 * 1024 * 1024,
        ),
    )(x, router_W, route_idx, expert_W, shared_W)


# device time: 264782 ns/iter; 1.2217x vs baseline; 1.2217x over previous
import jax
import jax.numpy as jnp
from jax import lax
from jax.experimental import pallas as pl
from jax.experimental.pallas import tpu as pltpu

N_DEV = 32
M = 2048
D = 512
H = 1024
E_LOCAL = 4
CH = M // N_DEV


def kernel(x, router_W, route_idx, expert_W, shared_W):
    my_out = lax.axis_index("i")
    shift = my_out * CH
    x_rot = jnp.roll(x, -shift, axis=0)
    idx_rot = jnp.roll(route_idx, -shift, axis=0)

    def body(x_ref, rw_ref, idx_ref, ew_ref, sw_ref, out_ref,
             prot_ref, p1_buf, p1_send, p1_recv, p2_send, p2_recv):
        my = lax.axis_index("i")

        barrier = pltpu.get_barrier_semaphore()
        for k in range(1, N_DEV):
            pl.semaphore_signal(
                barrier, inc=1,
                device_id=((my + k) % N_DEV,),
                device_id_type=pl.DeviceIdType.MESH,
            )
        pl.semaphore_wait(barrier, N_DEV - 1)

        xv = x_ref[:, :]
        scores = jnp.dot(xv, rw_ref[:, :], preferred_element_type=jnp.float32)
        s_max = jnp.max(scores, axis=-1, keepdims=True)
        ex = jnp.exp(scores - s_max)
        probs = ex / jnp.sum(ex, axis=-1, keepdims=True)
        route = idx_ref[:, :]
        col_ids = lax.broadcasted_iota(jnp.int32, scores.shape, 1)

        for e in range(E_LOCAL):
            ge = my * E_LOCAL + e
            g = jnp.sum(jnp.where(col_ids == ge, probs, 0.0),
                        axis=1, keepdims=True)
            g = jnp.where(route == ge, g, 0.0)
            contrib = jnp.dot(xv * g, ew_ref[e],
                              preferred_element_type=jnp.float32)
            if e == 0:
                prot_ref[:, :] = contrib
            else:
                prot_ref[:, :] = prot_ref[:, :] + contrib

        p1_sends = []
        for k in range(1, N_DEV):
            rdma = pltpu.make_async_remote_copy(
                src_ref=prot_ref.at[pl.ds(k * CH, CH), :],
                dst_ref=p1_buf.at[k - 1],
                send_sem=p1_send.at[k - 1],
                recv_sem=p1_recv.at[k - 1],
                device_id=((my + k) % N_DEV,),
                device_id_type=pl.DeviceIdType.MESH,
            )
            rdma.start()
            p1_sends.append(rdma)

        acc_o = (
            prot_ref[0:CH, :]
            + jnp.dot(x_ref[0:CH, :], sw_ref[:, :],
                      preferred_element_type=jnp.float32)
        )
        for j in range(N_DEV - 1):
            recv = pltpu.make_async_remote_copy(
                src_ref=prot_ref.at[pl.ds(0, CH), :],
                dst_ref=p1_buf.at[j],
                send_sem=p1_send.at[j],
                recv_sem=p1_recv.at[j],
                device_id=(my,),
                device_id_type=pl.DeviceIdType.MESH,
            )
            recv.wait_recv()
            acc_o = acc_o + p1_buf[j, :, :]
        prot_ref[0:CH, :] = acc_o
        out_ref[pl.ds(my * CH, CH), :] = acc_o

        p2_sends = []
        for k in range(1, N_DEV):
            rdma = pltpu.make_async_remote_copy(
                src_ref=prot_ref.at[pl.ds(0, CH), :],
                dst_ref=prot_ref.at[pl.ds((N_DEV - k) * CH, CH), :],
                send_sem=p2_send.at[k - 1],
                recv_sem=p2_recv.at[k - 1],
                device_id=((my + k) % N_DEV,),
                device_id_type=pl.DeviceIdType.MESH,
            )
            rdma.start()
            p2_sends.append(rdma)

        for j in range(N_DEV - 1):
            s_blk = N_DEV - 1 - j
            recv = pltpu.make_async_remote_copy(
                src_ref=prot_ref.at[pl.ds(0, CH), :],
                dst_ref=prot_ref.at[pl.ds(s_blk * CH, CH), :],
                send_sem=p2_send.at[j],
                recv_sem=p2_recv.at[j],
                device_id=(my,),
                device_id_type=pl.DeviceIdType.MESH,
            )
            recv.wait_recv()
            c = (my + s_blk) % N_DEV
            out_ref[pl.ds(c * CH, CH), :] = prot_ref[pl.ds(s_blk * CH, CH), :]

        for rdma in p1_sends:
            rdma.wait_send()
        for rdma in p2_sends:
            rdma.wait_send()

    return pl.pallas_call(
        body,
        out_shape=jax.ShapeDtypeStruct((M, H), jnp.float32),
        in_specs=[pl.BlockSpec(memory_space=pltpu.VMEM)] * 5,
        out_specs=pl.BlockSpec(memory_space=pltpu.VMEM),
        scratch_shapes=[
            pltpu.VMEM((M, H), jnp.float32),
            pltpu.VMEM((N_DEV - 1, CH, H), jnp.float32),
            pltpu.SemaphoreType.DMA((N_DEV - 1,)),
            pltpu.SemaphoreType.DMA((N_DEV - 1,)),
            pltpu.SemaphoreType.DMA((N_DEV - 1,)),
            pltpu.SemaphoreType.DMA((N_DEV - 1,)),
        ],
        compiler_params=pltpu.CompilerParams(
            collective_id=0, vmem_limit_bytes=100 * 1024 * 1024,
        ),
    )(x_rot, router_W, idx_rot, expert_W, shared_W)


# device time: 187216 ns/iter; 1.7278x vs baseline; 1.4143x over previous
import jax
import jax.numpy as jnp
from jax import lax
from jax.experimental import pallas as pl
from jax.experimental.pallas import tpu as pltpu

N_DEV = 32
M = 2048
D = 512
H = 1024
E_LOCAL = 4
CH = M // N_DEV
CAP = 16
HP = H + 128


def kernel(x, router_W, route_idx, expert_W, shared_W):
    my_out = lax.axis_index("i")
    shift = my_out * CH
    x_rot = jnp.roll(x, -shift, axis=0)
    idx_rot = jnp.roll(route_idx, -shift, axis=0)
    idx2d = jnp.reshape(idx_rot, (N_DEV, CH))

    def body(x_ref, rw_ref, idx_ref, idx2d_ref, ew_ref, sw_ref, out_ref,
             prot_ref, stage_ref, p1_buf, p1_send, p1_recv, p2_send, p2_recv):
        my = lax.axis_index("i")

        barrier = pltpu.get_barrier_semaphore()
        for k in range(1, N_DEV):
            pl.semaphore_signal(
                barrier, inc=1,
                device_id=((my + k) % N_DEV,),
                device_id_type=pl.DeviceIdType.MESH,
            )
        pl.semaphore_wait(barrier, N_DEV - 1)

        xv = x_ref[:, :]
        scores = jnp.dot(xv, rw_ref[:, :], preferred_element_type=jnp.float32)
        s_max = jnp.max(scores, axis=-1, keepdims=True)
        ex = jnp.exp(scores - s_max)
        probs = ex / jnp.sum(ex, axis=-1, keepdims=True)
        route = idx_ref[:, :]
        col_ids = lax.broadcasted_iota(jnp.int32, scores.shape, 1)

        for e in range(E_LOCAL):
            ge = my * E_LOCAL + e
            g = jnp.sum(jnp.where(col_ids == ge, probs, 0.0),
                        axis=1, keepdims=True)
            g = jnp.where(route == ge, g, 0.0)
            contrib = jnp.dot(xv * g, ew_ref[e],
                              preferred_element_type=jnp.float32)
            if e == 0:
                prot_ref[:, :] = contrib
            else:
                prot_ref[:, :] = prot_ref[:, :] + contrib

        maskR = (idx2d_ref[:, :] // E_LOCAL) == my
        maskF = maskR.astype(jnp.float32)
        r_i = lax.broadcasted_iota(jnp.int32, (CH, CH), 0)
        c_i = lax.broadcasted_iota(jnp.int32, (CH, CH), 1)
        upper = (r_i <= c_i).astype(jnp.float32)
        pcum = jnp.dot(maskF, upper, preferred_element_type=jnp.float32)
        r_row = lax.broadcasted_iota(jnp.int32, (1, CH), 1).astype(jnp.float32)
        c_col = lax.broadcasted_iota(jnp.int32, (CAP, 1), 0).astype(jnp.float32)

        p1_sends = []
        for k in range(1, N_DEV):
            mk = maskF[k:k + 1, :]
            pk = pcum[k:k + 1, :] - 1.0
            sel = jnp.where((pk == c_col) & (mk > 0), 1.0, 0.0)
            vals = jnp.dot(sel, prot_ref[pl.ds(k * CH, CH), :],
                           preferred_element_type=jnp.float32)
            cnt = jnp.sum(sel, axis=1, keepdims=True)
            idxv = jnp.sum(sel * r_row, axis=1, keepdims=True)
            idxv = jnp.where(cnt > 0, idxv, -1.0)
            stage_ref[k - 1, :, 0:H] = vals
            stage_ref[k - 1, :, H:H + 1] = idxv
            rdma = pltpu.make_async_remote_copy(
                src_ref=stage_ref.at[k - 1],
                dst_ref=p1_buf.at[k - 1],
                send_sem=p1_send.at[k - 1],
                recv_sem=p1_recv.at[k - 1],
                device_id=((my + k) % N_DEV,),
                device_id_type=pl.DeviceIdType.MESH,
            )
            rdma.start()
            p1_sends.append(rdma)

        acc_o = (
            prot_ref[0:CH, :]
            + jnp.dot(x_ref[0:CH, :], sw_ref[:, :],
                      preferred_element_type=jnp.float32)
        )
        r_rowF = lax.broadcasted_iota(jnp.int32, (1, CH), 1).astype(jnp.float32)
        for j in range(N_DEV - 1):
            recv = pltpu.make_async_remote_copy(
                src_ref=stage_ref.at[0],
                dst_ref=p1_buf.at[j],
                send_sem=p1_send.at[j],
                recv_sem=p1_recv.at[j],
                device_id=(my,),
                device_id_type=pl.DeviceIdType.MESH,
            )
            recv.wait_recv()
            vals = p1_buf[j, :, 0:H]
            idxc = p1_buf[j, :, H:H + 1]
            oh = jnp.where((idxc == r_rowF) & (idxc >= 0), 1.0, 0.0)
            add = lax.dot_general(
                oh, vals, dimension_numbers=(((0,), (0,)), ((), ())),
                preferred_element_type=jnp.float32)
            acc_o = acc_o + add
        prot_ref[0:CH, :] = acc_o

        p2_sends = []
        for k in range(1, N_DEV):
            rdma = pltpu.make_async_remote_copy(
                src_ref=prot_ref.at[pl.ds(0, CH), :],
                dst_ref=prot_ref.at[pl.ds((N_DEV - k) * CH, CH), :],
                send_sem=p2_send.at[k - 1],
                recv_sem=p2_recv.at[k - 1],
                device_id=((my + k) % N_DEV,),
                device_id_type=pl.DeviceIdType.MESH,
            )
            rdma.start()
            p2_sends.append(rdma)
        out_ref[pl.ds(my * CH, CH), :] = acc_o

        for j in range(N_DEV - 1):
            s_blk = N_DEV - 1 - j
            recv = pltpu.make_async_remote_copy(
                src_ref=prot_ref.at[pl.ds(0, CH), :],
                dst_ref=prot_ref.at[pl.ds(s_blk * CH, CH), :],
                send_sem=p2_send.at[j],
                recv_sem=p2_recv.at[j],
                device_id=(my,),
                device_id_type=pl.DeviceIdType.MESH,
            )
            recv.wait_recv()
            c = (my + s_blk) % N_DEV
            out_ref[pl.ds(c * CH, CH), :] = prot_ref[pl.ds(s_blk * CH, CH), :]

        for rdma in p1_sends:
            rdma.wait_send()
        for rdma in p2_sends:
            rdma.wait_send()

    return pl.pallas_call(
        body,
        out_shape=jax.ShapeDtypeStruct((M, H), jnp.float32),
        in_specs=[pl.BlockSpec(memory_space=pltpu.VMEM)] * 6,
        out_specs=pl.BlockSpec(memory_space=pltpu.VMEM),
        scratch_shapes=[
            pltpu.VMEM((M, H), jnp.float32),
            pltpu.VMEM((N_DEV - 1, CAP, HP), jnp.float32),
            pltpu.VMEM((N_DEV - 1, CAP, HP), jnp.float32),
            pltpu.SemaphoreType.DMA((N_DEV - 1,)),
            pltpu.SemaphoreType.DMA((N_DEV - 1,)),
            pltpu.SemaphoreType.DMA((N_DEV - 1,)),
            pltpu.SemaphoreType.DMA((N_DEV - 1,)),
        ],
        compiler_params=pltpu.CompilerParams(
            collective_id=0, vmem_limit_bytes=100 * 1024 * 1024,
        ),
    )(x_rot, router_W, idx_rot, idx2d, expert_W, shared_W)
